# Initial kernel scaffold; baseline (speedup 1.0000x reference)
import jax
import jax.numpy as jnp
from jax.experimental import pallas as pl


def kernel(x, Fy, F_x, st_rows, st_cols, st_vals, ts_rows, ts_cols, ts_vals, mu):
    raise NotImplementedError("write your pallas kernel here")


# speedup vs baseline: 8.3436x; 8.3436x over previous
import functools

import jax
import jax.numpy as jnp
from jax import lax
from jax.experimental import pallas as pl
from jax.experimental.pallas import tpu as pltpu
from jax.experimental.pallas import tpu_sc as plsc

N_SQ = 65536
N_TRI = 4096
NNZ = 262144
B = 16

NC = 2
NS = 16
NW = NC * NS
NNZ_W = NNZ // NW
CHUNK = 2048
DESC = 128
ND = CHUNK // DESC
NCHUNK = NNZ_W // CHUNK


def _spmm_body(n_out, src_hbm, cols_hbm, rows_hbm, vals_hbm, out_hbm,
               cols_v, rows_v, vals_v, gbuf, acc, sem):
    c = lax.axis_index("c")
    s = lax.axis_index("s")
    wid = c * NS + s

    def _zero_body(i, carry):
        gbuf[i, :] = jnp.zeros((16,), jnp.float32)
        return carry
    lax.fori_loop(0, CHUNK, _zero_body, None)

    rows_per_tile = n_out // NS
    zrows = min(rows_per_tile, CHUNK)
    for k in range(0, rows_per_tile, zrows):
        pltpu.sync_copy(gbuf.at[pl.ds(0, zrows)],
                        acc.at[pl.ds(s * rows_per_tile + k, zrows)])
    plsc.subcore_barrier()

    for ck in range(NCHUNK):
        off = pl.multiple_of(wid * NNZ_W + ck * CHUNK, CHUNK)
        dsc = pl.multiple_of(off // DESC, ND)
        pltpu.sync_copy(cols_hbm.at[pl.ds(dsc, ND)], cols_v)
        pltpu.sync_copy(rows_hbm.at[pl.ds(dsc, ND)], rows_v)
        pltpu.sync_copy(vals_hbm.at[pl.ds(off, CHUNK)], vals_v)
        handles = [
            pltpu.async_copy(src_hbm.at[cols_v.at[j]],
                             gbuf.at[pl.ds(j * DESC, DESC)], sem)
            for j in range(ND)
        ]
        for h in handles:
            h.wait()

        def _scale_body(k, carry):
            base = pl.multiple_of(k * 16, 16)
            v = vals_v[pl.ds(base, 16)]
            for j in range(16):
                gbuf[base + j, :] = gbuf[base + j, :] * v[j]
            return carry
        lax.fori_loop(0, CHUNK // 16, _scale_body, None)

        for j in range(ND):
            pltpu.sync_copy(gbuf.at[pl.ds(j * DESC, DESC)],
                            acc.at[rows_v.at[j]], add=True)

    plsc.subcore_barrier()
    pltpu.sync_copy(acc.at[pl.ds(s * rows_per_tile, rows_per_tile)],
                    out_hbm.at[c, pl.ds(s * rows_per_tile, rows_per_tile)])


def _spmm_sc(srcT, cols2d, rows2d, vals, n_out):
    mesh = plsc.VectorSubcoreMesh(core_axis_name="c", subcore_axis_name="s",
                                  num_cores=NC, num_subcores=NS)
    kern = pl.kernel(
        functools.partial(_spmm_body, n_out),
        out_type=jax.ShapeDtypeStruct((NC, n_out, B), jnp.float32),
        mesh=mesh,
        scratch_types=[
            pltpu.VMEM((ND, DESC), jnp.int32),
            pltpu.VMEM((ND, DESC), jnp.int32),
            pltpu.VMEM((CHUNK,), jnp.float32),
            pltpu.VMEM((CHUNK, B), jnp.float32),
            pltpu.VMEM_SHARED((n_out, B), jnp.float32),
            pltpu.SemaphoreType.DMA,
        ],
        compiler_params=pltpu.CompilerParams(use_tc_tiling_on_sc=False),
    )
    return kern(srcT, cols2d, rows2d, vals)


def _matmul_body(p0_ref, p1_ref, fx_ref, out_ref):
    xt = p0_ref[...] + p1_ref[...]
    out_ref[...] = jnp.dot(fx_ref[...], xt,
                           preferred_element_type=jnp.float32)


def _matmul_tc(parts, F_x):
    bm = 512
    grid = (N_TRI // bm,)
    return pl.pallas_call(
        _matmul_body,
        grid=grid,
        in_specs=[
            pl.BlockSpec((N_TRI, B), lambda i: (0, 0)),
            pl.BlockSpec((N_TRI, B), lambda i: (0, 0)),
            pl.BlockSpec((bm, N_TRI), lambda i: (i, 0)),
        ],
        out_specs=pl.BlockSpec((bm, B), lambda i: (i, 0)),
        out_shape=jax.ShapeDtypeStruct((N_TRI, B), jnp.float32),
    )(parts[0], parts[1], F_x)


def _final_body(mu_ref, xT_ref, fyT_ref, p0_ref, p1_ref, out_ref):
    mu = mu_ref[0, 0]
    fx = p0_ref[...] + p1_ref[...]
    z = xT_ref[...] + mu * (fyT_ref[...] - fx)
    out_ref[...] = jnp.maximum(z, 0.0)


def _final_tc(mu, xT, FyT, parts):
    rows = N_SQ * B // 128
    bm = 1024
    grid = (rows // bm,)
    x2 = xT.reshape(rows, 128)
    fy2 = FyT.reshape(rows, 128)
    p02 = parts[0].reshape(rows, 128)
    p12 = parts[1].reshape(rows, 128)
    mu2 = jnp.asarray(mu, jnp.float32).reshape(1, 1)
    zT = pl.pallas_call(
        _final_body,
        grid=grid,
        in_specs=[
            pl.BlockSpec(memory_space=pltpu.SMEM),
            pl.BlockSpec((bm, 128), lambda i: (i, 0)),
            pl.BlockSpec((bm, 128), lambda i: (i, 0)),
            pl.BlockSpec((bm, 128), lambda i: (i, 0)),
            pl.BlockSpec((bm, 128), lambda i: (i, 0)),
        ],
        out_specs=pl.BlockSpec((bm, 128), lambda i: (i, 0)),
        out_shape=jax.ShapeDtypeStruct((rows, 128), jnp.float32),
    )(mu2, x2, fy2, p02, p12)
    return zT.reshape(N_SQ, B)


def kernel(x, Fy, F_x, st_rows, st_cols, st_vals, ts_rows, ts_cols, ts_vals, mu):
    xT = x.T.reshape(N_SQ, B)
    FyT = Fy.T.reshape(N_SQ, B)
    st_cols2d = st_cols.astype(jnp.int32).reshape(NNZ // DESC, DESC)
    st_rows2d = st_rows.astype(jnp.int32).reshape(NNZ // DESC, DESC)
    ts_cols2d = ts_cols.astype(jnp.int32).reshape(NNZ // DESC, DESC)
    ts_rows2d = ts_rows.astype(jnp.int32).reshape(NNZ // DESC, DESC)

    xt_parts = _spmm_sc(xT, st_cols2d, st_rows2d, st_vals, N_TRI)
    fxtT = _matmul_tc(xt_parts, F_x)
    fx_parts = _spmm_sc(fxtT, ts_cols2d, ts_rows2d, ts_vals, N_SQ)
    zT = _final_tc(mu, xT, FyT, fx_parts)
    return zT.T
